# baseline (speedup 1.0000x reference)
import functools

import jax
import jax.numpy as jnp
from jax import lax
from jax.experimental import pallas as pl
from jax.experimental.pallas import tpu as pltpu
from jax.experimental.pallas import tpu_sc as plsc

EMBED_DIM = 64
LANES = 128
NSLICES = 8


def _pick_nbuf(units_per_w: int) -> int:
    for nbuf in (4, 5, 2):
        if units_per_w % nbuf == 0 and units_per_w // nbuf >= 2:
            return nbuf
    raise ValueError(units_per_w)


@functools.lru_cache(maxsize=None)
def _make_sc_gather(n_units: int, d: int):
    info = plsc.get_sparse_core_info()
    nc, ns = info.num_cores, info.num_subcores
    nw = nc * ns
    units_per_w = n_units // nw
    assert units_per_w * nw == n_units
    NBUF = _pick_nbuf(units_per_w)
    mesh = plsc.VectorSubcoreMesh(core_axis_name="c", subcore_axis_name="s")

    @functools.partial(
        pl.kernel,
        mesh=mesh,
        out_type=jax.ShapeDtypeStruct((n_units, d, LANES), jnp.float32),
        scratch_types=[
            pltpu.VMEM((units_per_w, LANES), jnp.int32),
            pltpu.VMEM((NBUF, LANES, d), jnp.float32),
        ]
        + [pltpu.SemaphoreType.DMA] * (2 * NBUF),
        compiler_params=pltpu.CompilerParams(use_tc_tiling_on_sc=False),
    )
    def sc_gather(table_hbm, idx_hbm, out_hbm, idx_v, rows_v, *sems):
        wid = lax.axis_index("s") * nc + lax.axis_index("c")
        base = wid * units_per_w
        gsems = sems[:NBUF]
        wsems = sems[NBUF:]

        pltpu.sync_copy(idx_hbm.at[pl.ds(base, units_per_w)], idx_v)

        def start_gather(g, b):
            pltpu.async_copy(table_hbm.at[idx_v.at[g]], rows_v.at[b], gsems[b])

        def wait_gather(g, b):
            pltpu.make_async_copy(table_hbm.at[idx_v.at[g]],
                                  rows_v.at[b], gsems[b]).wait()

        def start_write(g, b):
            pltpu.async_copy(rows_v.at[b, pl.ds(0, d)],
                             out_hbm.at[base + g, :, pl.ds(0, d)], wsems[b])
            pltpu.async_copy(rows_v.at[b, pl.ds(d, d)],
                             out_hbm.at[base + g, :, pl.ds(d, d)], wsems[b])

        def wait_write(g, b):
            pltpu.make_async_copy(rows_v.at[b, pl.ds(0, d)],
                                  out_hbm.at[base + g, :, pl.ds(0, d)],
                                  wsems[b]).wait()
            pltpu.make_async_copy(rows_v.at[b, pl.ds(d, d)],
                                  out_hbm.at[base + g, :, pl.ds(d, d)],
                                  wsems[b]).wait()

        for b in range(NBUF):
            start_gather(b, b)

        def outer(t, carry):
            for b in range(NBUF):
                g = NBUF * t + b
                wait_gather(g, b)
                start_write(g, b)
                wait_write(g, b)
                start_gather(g + NBUF, b)
            return carry

        lax.fori_loop(0, units_per_w // NBUF - 1, outer, 0)

        for b in range(NBUF):
            g = units_per_w - NBUF + b
            wait_gather(g, b)
            start_write(g, b)
        for b in range(NBUF):
            g = units_per_w - NBUF + b
            wait_write(g, b)

    return sc_gather


def _pick_hb(hist_s: int) -> int:
    for hb in (8, 5, 4, 2, 1):
        if hist_s % hb == 0:
            return hb
    raise ValueError(hist_s)


@functools.lru_cache(maxsize=None)
def _make_tc_transpose(hist: int, hist_s: int, h0: int, kblocks: int,
                       aliased: bool):
    hb = _pick_hb(hist_s)

    def write_block(in_ref, out_ref):
        eye = (lax.broadcasted_iota(jnp.int32, (2 * EMBED_DIM,) * 2, 0)
               == lax.broadcasted_iota(jnp.int32, (2 * EMBED_DIM,) * 2, 1)
               ).astype(jnp.float32)
        for h in range(hb):
            y2 = in_ref[h * kblocks:(h + 1) * kblocks].reshape(
                kblocks * EMBED_DIM, 2 * EMBED_DIM)
            yt = lax.dot_general(eye, y2, (((1,), (1,)), ((), ())),
                                 preferred_element_type=jnp.float32)
            for u in range(kblocks):
                out_ref[h, :, u * LANES:u * LANES + EMBED_DIM] = (
                    yt[:EMBED_DIM, u * EMBED_DIM:(u + 1) * EMBED_DIM])
                out_ref[h, :, u * LANES + EMBED_DIM:(u + 1) * LANES] = (
                    yt[EMBED_DIM:, u * EMBED_DIM:(u + 1) * EMBED_DIM])

    if aliased:
        def body(in_ref, prev_ref, out_ref):
            del prev_ref
            write_block(in_ref, out_ref)
    else:
        body = write_block

    in_specs = [pl.BlockSpec((hb * kblocks, EMBED_DIM, LANES),
                             lambda h: (h, 0, 0))]
    if aliased:
        in_specs.append(pl.BlockSpec(memory_space=pl.ANY))
    return pl.pallas_call(
        body,
        grid=(hist_s // hb,),
        in_specs=in_specs,
        out_specs=pl.BlockSpec((hb, EMBED_DIM, kblocks * LANES),
                               lambda h: (h0 // hb + h, 0, 0)),
        out_shape=jax.ShapeDtypeStruct(
            (hist, EMBED_DIM, kblocks * LANES), jnp.float32),
        input_output_aliases={1: 0} if aliased else {},
        compiler_params=pltpu.CompilerParams(
            dimension_semantics=("parallel",)),
    )


def kernel(input_ids, table):
    batch, hist = input_ids.shape
    kblocks = batch // LANES
    n_units = hist * kblocks
    nsl = NSLICES if hist % NSLICES == 0 else 1
    hist_s = hist // nsl
    units_s = hist_s * kblocks
    ids_t = input_ids.astype(jnp.int32).T.reshape(n_units, LANES)
    sc = _make_sc_gather(units_s, EMBED_DIM)
    l = None
    for s in range(nsl):
        v_s = sc(table, ids_t[s * units_s:(s + 1) * units_s])
        tc = _make_tc_transpose(hist, hist_s, s * hist_s, kblocks, s > 0)
        l = tc(v_s) if s == 0 else tc(v_s, l)
    return l.transpose(2, 0, 1)


# speedup vs baseline: 1.0437x; 1.0437x over previous
import functools

import jax
import jax.numpy as jnp
from jax import lax
from jax.experimental import pallas as pl
from jax.experimental.pallas import tpu as pltpu
from jax.experimental.pallas import tpu_sc as plsc

EMBED_DIM = 64
LANES = 128
NSLICES = 1


def _pick_nbuf(units_per_w: int) -> int:
    for nbuf in (4, 5, 2):
        if units_per_w % nbuf == 0 and units_per_w // nbuf >= 2:
            return nbuf
    raise ValueError(units_per_w)


@functools.lru_cache(maxsize=None)
def _make_sc_gather(hist_s: int, batch: int, d: int):
    info = plsc.get_sparse_core_info()
    nc, ns = info.num_cores, info.num_subcores
    nw = nc * ns
    kblocks = batch // LANES
    assert kblocks == nw
    units_per_w = hist_s
    n_units = hist_s * kblocks
    NBUF = _pick_nbuf(units_per_w)
    mesh = plsc.VectorSubcoreMesh(core_axis_name="c", subcore_axis_name="s")

    @functools.partial(
        pl.kernel,
        mesh=mesh,
        out_type=jax.ShapeDtypeStruct((n_units, d, LANES), jnp.float32),
        scratch_types=[
            pltpu.VMEM((units_per_w, LANES), jnp.int32),
            pltpu.VMEM((NBUF, LANES, d), jnp.float32),
        ]
        + [pltpu.SemaphoreType.DMA] * (2 * NBUF),
        compiler_params=pltpu.CompilerParams(use_tc_tiling_on_sc=False),
    )
    def sc_gather(table_hbm, idx_hbm, out_hbm, idx_v, rows_v, *sems):
        wid = lax.axis_index("s") * nc + lax.axis_index("c")
        gsems = sems[:NBUF]
        wsems = sems[NBUF:]

        pltpu.sync_copy(idx_hbm.at[:, wid], idx_v)

        def start_gather(g, b):
            pltpu.async_copy(table_hbm.at[idx_v.at[g]], rows_v.at[b], gsems[b])

        def wait_gather(g, b):
            pltpu.make_async_copy(table_hbm.at[idx_v.at[g]],
                                  rows_v.at[b], gsems[b]).wait()

        def start_write(g, b):
            u = g * kblocks + wid
            pltpu.async_copy(rows_v.at[b, pl.ds(0, d)],
                             out_hbm.at[u, :, pl.ds(0, d)], wsems[b])
            pltpu.async_copy(rows_v.at[b, pl.ds(d, d)],
                             out_hbm.at[u, :, pl.ds(d, d)], wsems[b])

        def wait_write(g, b):
            u = g * kblocks + wid
            pltpu.make_async_copy(rows_v.at[b, pl.ds(0, d)],
                                  out_hbm.at[u, :, pl.ds(0, d)],
                                  wsems[b]).wait()
            pltpu.make_async_copy(rows_v.at[b, pl.ds(d, d)],
                                  out_hbm.at[u, :, pl.ds(d, d)],
                                  wsems[b]).wait()

        for b in range(NBUF):
            start_gather(b, b)

        def outer(t, carry):
            for b in range(NBUF):
                g = NBUF * t + b
                wait_gather(g, b)
                start_write(g, b)
                wait_write(g, b)
                start_gather(g + NBUF, b)
            return carry

        lax.fori_loop(0, units_per_w // NBUF - 1, outer, 0)

        for b in range(NBUF):
            g = units_per_w - NBUF + b
            wait_gather(g, b)
            start_write(g, b)
        for b in range(NBUF):
            g = units_per_w - NBUF + b
            wait_write(g, b)

    return sc_gather


def _pick_hb(hist_s: int) -> int:
    for hb in (8, 5, 4, 2, 1):
        if hist_s % hb == 0:
            return hb
    raise ValueError(hist_s)


@functools.lru_cache(maxsize=None)
def _make_tc_transpose(hist: int, hist_s: int, h0: int, kblocks: int,
                       aliased: bool):
    hb = _pick_hb(hist_s)

    def write_block(in_ref, out_ref):
        eye = (lax.broadcasted_iota(jnp.int32, (2 * EMBED_DIM,) * 2, 0)
               == lax.broadcasted_iota(jnp.int32, (2 * EMBED_DIM,) * 2, 1)
               ).astype(jnp.float32)
        for h in range(hb):
            y2 = in_ref[h * kblocks:(h + 1) * kblocks].reshape(
                kblocks * EMBED_DIM, 2 * EMBED_DIM)
            yt = lax.dot_general(eye, y2, (((1,), (1,)), ((), ())),
                                 preferred_element_type=jnp.float32)
            for u in range(kblocks):
                out_ref[h, :, u * LANES:u * LANES + EMBED_DIM] = (
                    yt[:EMBED_DIM, u * EMBED_DIM:(u + 1) * EMBED_DIM])
                out_ref[h, :, u * LANES + EMBED_DIM:(u + 1) * LANES] = (
                    yt[EMBED_DIM:, u * EMBED_DIM:(u + 1) * EMBED_DIM])

    if aliased:
        def body(in_ref, prev_ref, out_ref):
            del prev_ref
            write_block(in_ref, out_ref)
    else:
        body = write_block

    in_specs = [pl.BlockSpec((hb * kblocks, EMBED_DIM, LANES),
                             lambda h: (h, 0, 0))]
    if aliased:
        in_specs.append(pl.BlockSpec(memory_space=pl.ANY))
    return pl.pallas_call(
        body,
        grid=(hist_s // hb,),
        in_specs=in_specs,
        out_specs=pl.BlockSpec((hb, EMBED_DIM, kblocks * LANES),
                               lambda h: (h0 // hb + h, 0, 0)),
        out_shape=jax.ShapeDtypeStruct(
            (hist, EMBED_DIM, kblocks * LANES), jnp.float32),
        input_output_aliases={1: 0} if aliased else {},
        compiler_params=pltpu.CompilerParams(
            dimension_semantics=("parallel",),
            vmem_limit_bytes=50 * 1024 * 1024),
    )


@functools.lru_cache(maxsize=None)
def _make_tc_ids_t(batch: int, hist: int):
    kblocks = batch // LANES
    kb = 8

    def body(in_ref, out_ref):
        for j in range(kb):
            out_ref[:, j, :] = jnp.transpose(
                in_ref[j * LANES:(j + 1) * LANES, :])

    return pl.pallas_call(
        body,
        grid=(kblocks // kb,),
        in_specs=[pl.BlockSpec((kb * LANES, hist), lambda i: (i, 0))],
        out_specs=pl.BlockSpec((hist, kb, LANES), lambda i: (0, i, 0)),
        out_shape=jax.ShapeDtypeStruct((hist, kblocks, LANES), jnp.int32),
        compiler_params=pltpu.CompilerParams(
            dimension_semantics=("arbitrary",)),
    )


def kernel(input_ids, table):
    batch, hist = input_ids.shape
    kblocks = batch // LANES
    nsl = NSLICES if hist % NSLICES == 0 else 1
    hist_s = hist // nsl
    ids_t = _make_tc_ids_t(batch, hist)(input_ids.astype(jnp.int32))
    sc = _make_sc_gather(hist_s, batch, EMBED_DIM)
    l = None
    for s in range(nsl):
        v_s = sc(table, ids_t[s * hist_s:(s + 1) * hist_s])
        tc = _make_tc_transpose(hist, hist_s, s * hist_s, kblocks, s > 0)
        l = tc(v_s) if s == 0 else tc(v_s, l)
    return l.transpose(2, 0, 1)
